# baseline (speedup 1.0000x reference)
import functools

import jax
import jax.numpy as jnp
from jax import lax
from jax.experimental import pallas as pl
from jax.experimental.pallas import tpu as pltpu
from jax.experimental.pallas import tpu_sc as plsc


def _make_copy_kernel(S, D, num_cores, num_subcores):
    nw = num_cores * num_subcores
    rows_per_w = S // nw
    mesh = plsc.VectorSubcoreMesh(core_axis_name="c", subcore_axis_name="s")

    n_chunks = 4
    c_rows = rows_per_w // n_chunks

    @functools.partial(
        pl.kernel,
        mesh=mesh,
        out_type=jax.ShapeDtypeStruct((S, D), jnp.float32),
        scratch_types=[
            pltpu.VMEM((n_chunks, c_rows, D), jnp.float32),
            pltpu.SemaphoreType.DMA((n_chunks,)),
            pltpu.SemaphoreType.DMA((n_chunks,)),
        ],
    )
    def copy_k(w_hbm, out_hbm, buf, in_sems, out_sems):
        wid = lax.axis_index("s") * num_cores + lax.axis_index("c")
        base = wid * rows_per_w
        loads = []
        for i in range(n_chunks):
            loads.append(
                pltpu.async_copy(
                    w_hbm.at[pl.ds(base + i * c_rows, c_rows)],
                    buf.at[i],
                    in_sems.at[i],
                )
            )
        stores = []
        for i in range(n_chunks):
            loads[i].wait()
            stores.append(
                pltpu.async_copy(
                    buf.at[i],
                    out_hbm.at[pl.ds(base + i * c_rows, c_rows)],
                    out_sems.at[i],
                )
            )
        for s in stores:
            s.wait()

    return copy_k


def _tc_copy(w):
    S, D = w.shape
    n_chunks = 8
    c_rows = S // n_chunks

    def body(w_ref, o_ref, sems):
        copies = []
        for i in range(n_chunks):
            copies.append(
                pltpu.make_async_copy(
                    w_ref.at[pl.ds(i * c_rows, c_rows)],
                    o_ref.at[pl.ds(i * c_rows, c_rows)],
                    sems.at[i],
                )
            )
            copies[-1].start()
        for c in copies:
            c.wait()

    return pl.pallas_call(
        body,
        in_specs=[pl.BlockSpec(memory_space=pltpu.HBM)],
        out_specs=pl.BlockSpec(memory_space=pltpu.HBM),
        scratch_shapes=[pltpu.SemaphoreType.DMA((n_chunks,))],
        out_shape=jax.ShapeDtypeStruct((S, D), jnp.float32),
    )(w)


def kernel(x, embed_weight):
    S, D = embed_weight.shape
    seq_len = x.shape[1]
    out = _tc_copy(embed_weight)
    return out[None, :seq_len, :]


# speedup vs baseline: 34.1083x; 34.1083x over previous
import functools

import jax
import jax.numpy as jnp
from jax import lax
from jax.experimental import pallas as pl
from jax.experimental.pallas import tpu as pltpu
from jax.experimental.pallas import tpu_sc as plsc


def _make_copy_kernel(S, D, num_cores, num_subcores):
    nw = num_cores * num_subcores
    rows_per_w = S // nw
    mesh = plsc.VectorSubcoreMesh(core_axis_name="c", subcore_axis_name="s")

    n_chunks = 4
    c_rows = rows_per_w // n_chunks

    @functools.partial(
        pl.kernel,
        mesh=mesh,
        out_type=jax.ShapeDtypeStruct((S, D), jnp.float32),
        scratch_types=[
            pltpu.VMEM((n_chunks, c_rows, D), jnp.float32),
            pltpu.SemaphoreType.DMA((n_chunks,)),
            pltpu.SemaphoreType.DMA((n_chunks,)),
        ],
    )
    def copy_k(w_hbm, out_hbm, buf, in_sems, out_sems):
        wid = lax.axis_index("s") * num_cores + lax.axis_index("c")
        base = wid * rows_per_w
        loads = []
        for i in range(n_chunks):
            loads.append(
                pltpu.async_copy(
                    w_hbm.at[pl.ds(base + i * c_rows, c_rows)],
                    buf.at[i],
                    in_sems.at[i],
                )
            )
        stores = []
        for i in range(n_chunks):
            loads[i].wait()
            stores.append(
                pltpu.async_copy(
                    buf.at[i],
                    out_hbm.at[pl.ds(base + i * c_rows, c_rows)],
                    out_sems.at[i],
                )
            )
        for s in stores:
            s.wait()

    return copy_k


def _tc_copy(w, blk):
    S, D = w.shape

    def body(w_ref, o_ref):
        o_ref[...] = w_ref[...]

    return pl.pallas_call(
        body,
        grid=(S // blk,),
        in_specs=[pl.BlockSpec((blk, D), lambda i: (i, 0))],
        out_specs=pl.BlockSpec((blk, D), lambda i: (i, 0)),
        out_shape=jax.ShapeDtypeStruct((S, D), jnp.float32),
    )(w)


def kernel(x, embed_weight):
    S, D = embed_weight.shape
    seq_len = x.shape[1]
    out = _tc_copy(embed_weight, 512)
    return out[None, :seq_len, :]


# speedup vs baseline: 43.0404x; 1.2619x over previous
import functools

import jax
import jax.numpy as jnp
from jax import lax
from jax.experimental import pallas as pl
from jax.experimental.pallas import tpu as pltpu
from jax.experimental.pallas import tpu_sc as plsc


def _make_copy_kernel(S, D, num_cores, num_subcores):
    nw = num_cores * num_subcores
    rows_per_w = S // nw
    mesh = plsc.VectorSubcoreMesh(core_axis_name="c", subcore_axis_name="s")

    n_chunks = 4
    c_rows = rows_per_w // n_chunks

    @functools.partial(
        pl.kernel,
        mesh=mesh,
        out_type=jax.ShapeDtypeStruct((S, D), jnp.float32),
        scratch_types=[
            pltpu.VMEM((n_chunks, c_rows, D), jnp.float32),
            pltpu.SemaphoreType.DMA((n_chunks,)),
            pltpu.SemaphoreType.DMA((n_chunks,)),
        ],
    )
    def copy_k(w_hbm, out_hbm, buf, in_sems, out_sems):
        wid = lax.axis_index("s") * num_cores + lax.axis_index("c")
        base = wid * rows_per_w
        loads = []
        for i in range(n_chunks):
            loads.append(
                pltpu.async_copy(
                    w_hbm.at[pl.ds(base + i * c_rows, c_rows)],
                    buf.at[i],
                    in_sems.at[i],
                )
            )
        stores = []
        for i in range(n_chunks):
            loads[i].wait()
            stores.append(
                pltpu.async_copy(
                    buf.at[i],
                    out_hbm.at[pl.ds(base + i * c_rows, c_rows)],
                    out_sems.at[i],
                )
            )
        for s in stores:
            s.wait()

    return copy_k


def _tc_copy(w, blk):
    S, D = w.shape

    def body(w_ref, o_ref):
        o_ref[...] = w_ref[...]

    return pl.pallas_call(
        body,
        grid=(S // blk,),
        in_specs=[pl.BlockSpec((blk, D), lambda i: (i, 0))],
        out_specs=pl.BlockSpec((blk, D), lambda i: (i, 0)),
        out_shape=jax.ShapeDtypeStruct((S, D), jnp.float32),
    )(w)


def kernel(x, embed_weight):
    S, D = embed_weight.shape
    seq_len = x.shape[1]
    out = _tc_copy(embed_weight, 1024)
    return out[None, :seq_len, :]
